# baseline (speedup 1.0000x reference)
import functools

import jax
import jax.numpy as jnp
from jax import lax
from jax.experimental import pallas as pl
from jax.experimental.pallas import tpu as pltpu
from jax.experimental.pallas import tpu_sc as plsc

VOCAB = 100000
EMB = 300
HID = 100
OUT = 5
B = 4096
L = 200

DPAD = 128
PACK = 64
NLANE = 16
NW = 32
ROWS_PER_W = B // NW
C0, C1 = 104, 96
NSLOT = 4


def _mm_body(x_ref, w_ref, o_ref):
    t = jax.lax.dot_general(
        x_ref[...], w_ref[...],
        (((0,), (0,)), ((), ())), preferred_element_type=jnp.float32)
    resid = t[:, :28] - t[:, :28].astype(jnp.bfloat16).astype(jnp.float32)
    t = jnp.concatenate([t[:, :HID], resid], axis=1)
    lo = jax.lax.bitcast_convert_type(
        t[:, :64].astype(jnp.bfloat16), jnp.uint16).astype(jnp.uint32)
    hi = jax.lax.bitcast_convert_type(
        t[:, 64:].astype(jnp.bfloat16), jnp.uint16).astype(jnp.uint32)
    packed = jax.lax.bitcast_convert_type(lo | (hi << 16), jnp.float32)
    h = packed.shape[0] // 2
    o_ref[...] = jnp.concatenate([packed[:h], packed[h:]], axis=1)


VB = 2048
NBLK = (VOCAB + VB - 1) // VB
VSLOTS = NBLK * VB


def _table_matmul(embed_t, w1pad):
    return pl.pallas_call(
        _mm_body,
        grid=(NBLK,),
        in_specs=[
            pl.BlockSpec((EMB, VB), lambda i: (0, i)),
            pl.BlockSpec((EMB, DPAD), lambda i: (0, 0)),
        ],
        out_specs=pl.BlockSpec((VB // 2, DPAD), lambda i: (i, 0)),
        out_shape=jax.ShapeDtypeStruct((VSLOTS // 2, DPAD), jnp.float32),
    )(embed_t, w1pad)


CHUNKS = ((0, 56), (56, 48), (104, 48), (152, 48))
CMAX = 56


def _sc_body(idx_hbm, tab_hbm, out_hbm, idx_v, b0, b1, b2, b3, b4, b5, b6, b7,
             out_v, s0, s1, s2, s3, s4, s5, s6, s7):
    wid = lax.axis_index("s") * 2 + lax.axis_index("c")
    base = wid * ROWS_PER_W
    pltpu.sync_copy(idx_hbm.at[pl.ds(base * L, ROWS_PER_W * L)], idx_v)
    slots = ((b0, s0), (b1, s1), (b2, s2), (b3, s3),
             (b4, s4), (b5, s5), (b6, s6), (b7, s7))

    def xform(k, carry):
        v = idx_v[pl.ds(k * NLANE, NLANE)]
        j = v & (VB - 1)
        idx_v[pl.ds(k * NLANE, NLANE)] = (
            v - j + ((j & (VB // 2 - 1)) << 1) + (j >> 10))
        return carry

    lax.fori_loop(0, ROWS_PER_W * L // NLANE, xform, 0, unroll=8)

    def fire(r, q, buf, sem):
        off, n = CHUNKS[q]
        o = pl.multiple_of(r * L + off, 8)
        pltpu.async_copy(tab_hbm.at[idx_v.at[pl.ds(o, n)]],
                         buf.at[pl.ds(0, n)], sem)

    def wait(r, q, buf, sem):
        off, n = CHUNKS[q]
        o = pl.multiple_of(r * L + off, 8)
        pltpu.make_async_copy(tab_hbm.at[idx_v.at[pl.ds(o, n)]],
                              buf.at[pl.ds(0, n)], sem).wait()

    def accum(buf, n, accs):
        def body(j, a):
            new = []
            for c in range(4):
                w = plsc.bitcast(buf[j, pl.ds(c * NLANE, NLANE)], jnp.bfloat16)
                e, o = plsc.unpack(w, format=plsc.PackFormat.INTERLEAVED,
                                   preferred_element_type=jnp.float32)
                new.append(a[c] + e)
                new.append(a[4 + c] + o)
            return tuple(new[0::2]) + tuple(new[1::2])
        return lax.fori_loop(0, n, body, accs, unroll=8)

    zero = tuple(jnp.zeros((NLANE,), jnp.float32) for _ in range(8))

    for q in range(4):
        fire(0, q, *slots[q])
        fire(1, q, *slots[4 + q])

    def outer(i, carry):
        r0 = 2 * i
        for r, s in ((r0, 0), (r0 + 1, 4)):
            accs = zero
            for q in range(4):
                buf, sem = slots[s + q]
                wait(r, q, buf, sem)
                accs = accum(buf, CHUNKS[q][1], accs)

                @pl.when(i + 1 < ROWS_PER_W // 2)
                def _():
                    fire(r + 2, q, buf, sem)

            for c in range(8):
                out_v[r, pl.ds(c * NLANE, NLANE)] = accs[c]
        return carry

    lax.fori_loop(0, ROWS_PER_W // 2, outer, 0)
    pltpu.sync_copy(out_v, out_hbm.at[pl.ds(base, ROWS_PER_W)])


def _sc_gather_sum(idx_flat, table):
    f = functools.partial(
        pl.kernel,
        out_type=jax.ShapeDtypeStruct((B, DPAD), jnp.float32),
        mesh=plsc.VectorSubcoreMesh(core_axis_name="c", subcore_axis_name="s"),
        compiler_params=pltpu.CompilerParams(use_tc_tiling_on_sc=False,
                                             needs_layout_passes=False),
        scratch_types=[
            pltpu.VMEM((ROWS_PER_W * L,), jnp.int32),
        ] + [pltpu.VMEM((CMAX, PACK), jnp.float32)] * 8 + [
            pltpu.VMEM((ROWS_PER_W, DPAD), jnp.float32),
        ] + [pltpu.SemaphoreType.DMA] * 8,
    )(_sc_body)
    return f(idx_flat, table.reshape(VSLOTS, PACK))


def _mlp_body(s_ref, b1_ref, w2_ref, b2_ref, w3_ref, b3_ref, o_ref):
    s = s_ref[...]
    s = jnp.concatenate([s[:, :28] + s[:, 100:], s[:, 28:100],
                         jnp.zeros_like(s[:, :28])], axis=1)
    h = jnp.tanh(s + b1_ref[...])
    h = jnp.tanh(jnp.dot(h, w2_ref[...], preferred_element_type=jnp.float32)
                 + b2_ref[...])
    o_ref[...] = (jnp.dot(h, w3_ref[...], preferred_element_type=jnp.float32)
                  + b3_ref[...])


def _mlp(s, b1p, w2p, b2p, w3p, b3p):
    bb = 512
    return pl.pallas_call(
        _mlp_body,
        grid=(B // bb,),
        in_specs=[
            pl.BlockSpec((bb, DPAD), lambda i: (i, 0)),
            pl.BlockSpec((1, DPAD), lambda i: (0, 0)),
            pl.BlockSpec((DPAD, DPAD), lambda i: (0, 0)),
            pl.BlockSpec((1, DPAD), lambda i: (0, 0)),
            pl.BlockSpec((DPAD, OUT), lambda i: (0, 0)),
            pl.BlockSpec((1, OUT), lambda i: (0, 0)),
        ],
        out_specs=pl.BlockSpec((bb, OUT), lambda i: (i, 0)),
        out_shape=jax.ShapeDtypeStruct((B, OUT), jnp.float32),
    )(s, b1p, w2p, b2p, w3p, b3p)


def kernel(inputs, embed, W1, b1, W2, b2, W3, b3):
    w1p = jnp.zeros((EMB, DPAD), jnp.float32).at[:, :HID].set(W1)
    table = _table_matmul(embed.T, w1p)
    s = _sc_gather_sum(inputs.reshape(-1), table)
    b1p = jnp.zeros((1, DPAD), jnp.float32).at[0, :HID].set(b1)
    w2p = jnp.zeros((DPAD, DPAD), jnp.float32).at[:HID, :HID].set(W2)
    b2p = jnp.zeros((1, DPAD), jnp.float32).at[0, :HID].set(b2)
    w3p = jnp.zeros((DPAD, OUT), jnp.float32).at[:HID, :].set(W3)
    b3p = b3.reshape(1, OUT)
    return _mlp(s, b1p, w2p, b2p, w3p, b3p)


# speedup vs baseline: 1.0003x; 1.0003x over previous
import functools

import jax
import jax.numpy as jnp
from jax import lax
from jax.experimental import pallas as pl
from jax.experimental.pallas import tpu as pltpu
from jax.experimental.pallas import tpu_sc as plsc

VOCAB = 100000
EMB = 300
HID = 100
OUT = 5
B = 4096
L = 200

DPAD = 128
PACK = 64
NLANE = 16
NW = 32
ROWS_PER_W = B // NW
C0, C1 = 104, 96
NSLOT = 4


def _mm_body(x_ref, w_ref, o_ref):
    t = jax.lax.dot_general(
        x_ref[...].astype(jnp.bfloat16), w_ref[...].astype(jnp.bfloat16),
        (((0,), (0,)), ((), ())), preferred_element_type=jnp.float32)
    resid = t[:, :28] - t[:, :28].astype(jnp.bfloat16).astype(jnp.float32)
    t = jnp.concatenate([t[:, :HID], resid], axis=1)
    lo = jax.lax.bitcast_convert_type(
        t[:, :64].astype(jnp.bfloat16), jnp.uint16).astype(jnp.uint32)
    hi = jax.lax.bitcast_convert_type(
        t[:, 64:].astype(jnp.bfloat16), jnp.uint16).astype(jnp.uint32)
    packed = jax.lax.bitcast_convert_type(lo | (hi << 16), jnp.float32)
    h = packed.shape[0] // 2
    o_ref[...] = jnp.concatenate([packed[:h], packed[h:]], axis=1)


VB = 2048
NBLK = (VOCAB + VB - 1) // VB
VSLOTS = NBLK * VB


def _table_matmul(embed_t, w1pad):
    return pl.pallas_call(
        _mm_body,
        grid=(NBLK,),
        in_specs=[
            pl.BlockSpec((EMB, VB), lambda i: (0, i)),
            pl.BlockSpec((EMB, DPAD), lambda i: (0, 0)),
        ],
        out_specs=pl.BlockSpec((VB // 2, DPAD), lambda i: (i, 0)),
        out_shape=jax.ShapeDtypeStruct((VSLOTS // 2, DPAD), jnp.float32),
    )(embed_t, w1pad)


CHUNKS = ((0, 56), (56, 48), (104, 48), (152, 48))
CMAX = 56


def _sc_body(idx_hbm, tab_hbm, out_hbm, idx_v, b0, b1, b2, b3, b4, b5, b6, b7,
             out_v, s0, s1, s2, s3, s4, s5, s6, s7):
    wid = lax.axis_index("s") * 2 + lax.axis_index("c")
    base = wid * ROWS_PER_W
    pltpu.sync_copy(idx_hbm.at[pl.ds(base * L, ROWS_PER_W * L)], idx_v)
    slots = ((b0, s0), (b1, s1), (b2, s2), (b3, s3),
             (b4, s4), (b5, s5), (b6, s6), (b7, s7))

    def xform(k, carry):
        v = idx_v[pl.ds(k * NLANE, NLANE)]
        j = v & (VB - 1)
        idx_v[pl.ds(k * NLANE, NLANE)] = (
            v - j + ((j & (VB // 2 - 1)) << 1) + (j >> 10))
        return carry

    lax.fori_loop(0, ROWS_PER_W * L // NLANE, xform, 0, unroll=8)

    def fire(r, q, buf, sem):
        off, n = CHUNKS[q]
        o = pl.multiple_of(r * L + off, 8)
        pltpu.async_copy(tab_hbm.at[idx_v.at[pl.ds(o, n)]],
                         buf.at[pl.ds(0, n)], sem)

    def wait(r, q, buf, sem):
        off, n = CHUNKS[q]
        o = pl.multiple_of(r * L + off, 8)
        pltpu.make_async_copy(tab_hbm.at[idx_v.at[pl.ds(o, n)]],
                              buf.at[pl.ds(0, n)], sem).wait()

    def accum(buf, n, accs):
        def body(j, a):
            new = []
            for c in range(4):
                w = plsc.bitcast(buf[j, pl.ds(c * NLANE, NLANE)], jnp.bfloat16)
                e, o = plsc.unpack(w, format=plsc.PackFormat.INTERLEAVED,
                                   preferred_element_type=jnp.float32)
                new.append(a[c] + e)
                new.append(a[4 + c] + o)
            return tuple(new[0::2]) + tuple(new[1::2])
        return lax.fori_loop(0, n, body, accs, unroll=8)

    zero = tuple(jnp.zeros((NLANE,), jnp.float32) for _ in range(8))

    for q in range(4):
        fire(0, q, *slots[q])
        fire(1, q, *slots[4 + q])

    def outer(i, carry):
        r0 = 2 * i
        for r, s in ((r0, 0), (r0 + 1, 4)):
            accs = zero
            for q in range(4):
                buf, sem = slots[s + q]
                wait(r, q, buf, sem)
                accs = accum(buf, CHUNKS[q][1], accs)

                @pl.when(i + 1 < ROWS_PER_W // 2)
                def _():
                    fire(r + 2, q, buf, sem)

            for c in range(8):
                out_v[r, pl.ds(c * NLANE, NLANE)] = accs[c]
        return carry

    lax.fori_loop(0, ROWS_PER_W // 2, outer, 0)
    pltpu.sync_copy(out_v, out_hbm.at[pl.ds(base, ROWS_PER_W)])


def _sc_gather_sum(idx_flat, table):
    f = functools.partial(
        pl.kernel,
        out_type=jax.ShapeDtypeStruct((B, DPAD), jnp.float32),
        mesh=plsc.VectorSubcoreMesh(core_axis_name="c", subcore_axis_name="s"),
        compiler_params=pltpu.CompilerParams(use_tc_tiling_on_sc=False,
                                             needs_layout_passes=False),
        scratch_types=[
            pltpu.VMEM((ROWS_PER_W * L,), jnp.int32),
        ] + [pltpu.VMEM((CMAX, PACK), jnp.float32)] * 8 + [
            pltpu.VMEM((ROWS_PER_W, DPAD), jnp.float32),
        ] + [pltpu.SemaphoreType.DMA] * 8,
    )(_sc_body)
    return f(idx_flat, table.reshape(VSLOTS, PACK))


def _mlp_body(s_ref, b1_ref, w2_ref, b2_ref, w3_ref, b3_ref, o_ref):
    s = s_ref[...]
    s = jnp.concatenate([s[:, :28] + s[:, 100:], s[:, 28:100],
                         jnp.zeros_like(s[:, :28])], axis=1)
    h = jnp.tanh(s + b1_ref[...])
    h = jnp.tanh(jnp.dot(h, w2_ref[...], preferred_element_type=jnp.float32)
                 + b2_ref[...])
    o_ref[...] = (jnp.dot(h, w3_ref[...], preferred_element_type=jnp.float32)
                  + b3_ref[...])


def _mlp(s, b1p, w2p, b2p, w3p, b3p):
    bb = 512
    return pl.pallas_call(
        _mlp_body,
        grid=(B // bb,),
        in_specs=[
            pl.BlockSpec((bb, DPAD), lambda i: (i, 0)),
            pl.BlockSpec((1, DPAD), lambda i: (0, 0)),
            pl.BlockSpec((DPAD, DPAD), lambda i: (0, 0)),
            pl.BlockSpec((1, DPAD), lambda i: (0, 0)),
            pl.BlockSpec((DPAD, OUT), lambda i: (0, 0)),
            pl.BlockSpec((1, OUT), lambda i: (0, 0)),
        ],
        out_specs=pl.BlockSpec((bb, OUT), lambda i: (i, 0)),
        out_shape=jax.ShapeDtypeStruct((B, OUT), jnp.float32),
    )(s, b1p, w2p, b2p, w3p, b3p)


def kernel(inputs, embed, W1, b1, W2, b2, W3, b3):
    w1p = jnp.zeros((EMB, DPAD), jnp.float32).at[:, :HID].set(W1)
    table = _table_matmul(embed.T, w1p)
    s = _sc_gather_sum(inputs.reshape(-1), table)
    b1p = jnp.zeros((1, DPAD), jnp.float32).at[0, :HID].set(b1)
    w2p = jnp.zeros((DPAD, DPAD), jnp.float32).at[:HID, :HID].set(W2)
    b2p = jnp.zeros((1, DPAD), jnp.float32).at[0, :HID].set(b2)
    w3p = jnp.zeros((DPAD, OUT), jnp.float32).at[:HID, :].set(W3)
    b3p = b3.reshape(1, OUT)
    return _mlp(s, b1p, w2p, b2p, w3p, b3p)


# speedup vs baseline: 1.0010x; 1.0007x over previous
import functools

import jax
import jax.numpy as jnp
from jax import lax
from jax.experimental import pallas as pl
from jax.experimental.pallas import tpu as pltpu
from jax.experimental.pallas import tpu_sc as plsc

VOCAB = 100000
EMB = 300
HID = 100
OUT = 5
B = 4096
L = 200

DPAD = 128
PACK = 64
NLANE = 16
NW = 32
ROWS_PER_W = B // NW
C0, C1 = 104, 96
NSLOT = 4


def _mm_body(x_ref, w_ref, o_ref):
    t = jax.lax.dot_general(
        x_ref[...], w_ref[...],
        (((0,), (0,)), ((), ())), preferred_element_type=jnp.float32)
    resid = t[:, :28] - t[:, :28].astype(jnp.bfloat16).astype(jnp.float32)
    t = jnp.concatenate([t[:, :HID], resid], axis=1)
    lo = jax.lax.bitcast_convert_type(
        t[:, :64].astype(jnp.bfloat16), jnp.uint16).astype(jnp.uint32)
    hi = jax.lax.bitcast_convert_type(
        t[:, 64:].astype(jnp.bfloat16), jnp.uint16).astype(jnp.uint32)
    packed = jax.lax.bitcast_convert_type(lo | (hi << 16), jnp.float32)
    h = packed.shape[0] // 2
    o_ref[...] = jnp.concatenate([packed[:h], packed[h:]], axis=1)


VB = 2048
NBLK = (VOCAB + VB - 1) // VB
VSLOTS = NBLK * VB


def _table_matmul(embed_t, w1pad):
    return pl.pallas_call(
        _mm_body,
        grid=(NBLK,),
        in_specs=[
            pl.BlockSpec((EMB, VB), lambda i: (0, i)),
            pl.BlockSpec((EMB, DPAD), lambda i: (0, 0)),
        ],
        out_specs=pl.BlockSpec((VB // 2, DPAD), lambda i: (i, 0)),
        out_shape=jax.ShapeDtypeStruct((VSLOTS // 2, DPAD), jnp.float32),
    )(embed_t, w1pad)


CHUNKS = ((0, 56), (56, 48), (104, 48), (152, 48))
CMAX = 56


def _sc_body(idx_hbm, tab_hbm, out_hbm, idx_v, b0, b1, b2, b3, b4, b5, b6, b7,
             out_v, s0, s1, s2, s3, s4, s5, s6, s7):
    wid = lax.axis_index("s") * 2 + lax.axis_index("c")
    base = wid * ROWS_PER_W
    pltpu.sync_copy(idx_hbm.at[pl.ds(base * L, ROWS_PER_W * L)], idx_v)
    slots = ((b0, s0), (b1, s1), (b2, s2), (b3, s3),
             (b4, s4), (b5, s5), (b6, s6), (b7, s7))

    def xform(k, carry):
        v = idx_v[pl.ds(k * NLANE, NLANE)]
        j = v & (VB - 1)
        idx_v[pl.ds(k * NLANE, NLANE)] = (
            v - j + ((j & (VB // 2 - 1)) << 1) + (j >> 10))
        return carry

    lax.fori_loop(0, ROWS_PER_W * L // NLANE, xform, 0, unroll=8)

    def fire(r, q, buf, sem):
        off, n = CHUNKS[q]
        o = pl.multiple_of(r * L + off, 8)
        pltpu.async_copy(tab_hbm.at[idx_v.at[pl.ds(o, n)]],
                         buf.at[pl.ds(0, n)], sem)

    def wait(r, q, buf, sem):
        off, n = CHUNKS[q]
        o = pl.multiple_of(r * L + off, 8)
        pltpu.make_async_copy(tab_hbm.at[idx_v.at[pl.ds(o, n)]],
                              buf.at[pl.ds(0, n)], sem).wait()

    def accum(buf, n, accs):
        def body(j, a):
            new = []
            for c in range(4):
                w = plsc.bitcast(buf[j, pl.ds(c * NLANE, NLANE)], jnp.bfloat16)
                e, o = plsc.unpack(w, format=plsc.PackFormat.INTERLEAVED,
                                   preferred_element_type=jnp.float32)
                new.append(a[c] + e)
                new.append(a[4 + c] + o)
            return tuple(new[0::2]) + tuple(new[1::2])
        return lax.fori_loop(0, n, body, accs, unroll=8)

    zero = tuple(jnp.zeros((NLANE,), jnp.float32) for _ in range(8))

    for q in range(4):
        fire(0, q, *slots[q])
        fire(1, q, *slots[4 + q])

    def outer(i, carry):
        r0 = 2 * i
        for r, s in ((r0, 0), (r0 + 1, 4)):
            accs = zero
            for q in range(4):
                buf, sem = slots[s + q]
                wait(r, q, buf, sem)
                accs = accum(buf, CHUNKS[q][1], accs)

                @pl.when(i + 1 < ROWS_PER_W // 2)
                def _():
                    fire(r + 2, q, buf, sem)

            for c in range(8):
                out_v[r, pl.ds(c * NLANE, NLANE)] = accs[c]
        return carry

    lax.fori_loop(0, ROWS_PER_W // 2, outer, 0)
    pltpu.sync_copy(out_v, out_hbm.at[pl.ds(base, ROWS_PER_W)])


def _sc_gather_sum(idx_flat, table):
    f = functools.partial(
        pl.kernel,
        out_type=jax.ShapeDtypeStruct((B, DPAD), jnp.float32),
        mesh=plsc.VectorSubcoreMesh(core_axis_name="c", subcore_axis_name="s"),
        compiler_params=pltpu.CompilerParams(use_tc_tiling_on_sc=False,
                                             needs_layout_passes=False),
        scratch_types=[
            pltpu.VMEM((ROWS_PER_W * L,), jnp.int32),
        ] + [pltpu.VMEM((CMAX, PACK), jnp.float32)] * 8 + [
            pltpu.VMEM((ROWS_PER_W, DPAD), jnp.float32),
        ] + [pltpu.SemaphoreType.DMA] * 8,
    )(_sc_body)
    return f(idx_flat, table.reshape(VSLOTS, PACK))


def _mlp_body(s_ref, b1_ref, w2_ref, b2_ref, w3_ref, b3_ref, o_ref):
    s = s_ref[...]
    s = jnp.concatenate([s[:, :28] + s[:, 100:], s[:, 28:100],
                         jnp.zeros_like(s[:, :28])], axis=1)
    h = jnp.tanh(s + b1_ref[...])
    h = jnp.tanh(jnp.dot(h, w2_ref[...], preferred_element_type=jnp.float32)
                 + b2_ref[...])
    o_ref[...] = (jnp.dot(h, w3_ref[...], preferred_element_type=jnp.float32)
                  + b3_ref[...])


def _mlp(s, b1p, w2p, b2p, w3p, b3p):
    bb = 512
    return pl.pallas_call(
        _mlp_body,
        grid=(B // bb,),
        in_specs=[
            pl.BlockSpec((bb, DPAD), lambda i: (i, 0)),
            pl.BlockSpec((1, DPAD), lambda i: (0, 0)),
            pl.BlockSpec((DPAD, DPAD), lambda i: (0, 0)),
            pl.BlockSpec((1, DPAD), lambda i: (0, 0)),
            pl.BlockSpec((DPAD, OUT), lambda i: (0, 0)),
            pl.BlockSpec((1, OUT), lambda i: (0, 0)),
        ],
        out_specs=pl.BlockSpec((bb, OUT), lambda i: (i, 0)),
        out_shape=jax.ShapeDtypeStruct((B, OUT), jnp.float32),
    )(s, b1p, w2p, b2p, w3p, b3p)


def kernel(inputs, embed, W1, b1, W2, b2, W3, b3):
    w1p = jnp.zeros((EMB, DPAD), jnp.float32).at[:, :HID].set(W1)
    table = _table_matmul(embed.T, w1p)
    s = _sc_gather_sum(inputs.reshape(-1), table)
    b1p = jnp.zeros((1, DPAD), jnp.float32).at[0, :HID].set(b1)
    w2p = jnp.zeros((DPAD, DPAD), jnp.float32).at[:HID, :HID].set(W2)
    b2p = jnp.zeros((1, DPAD), jnp.float32).at[0, :HID].set(b2)
    w3p = jnp.zeros((DPAD, OUT), jnp.float32).at[:HID, :].set(W3)
    b3p = b3.reshape(1, OUT)
    return _mlp(s, b1p, w2p, b2p, w3p, b3p)


# speedup vs baseline: 1.0682x; 1.0671x over previous
import functools

import jax
import jax.numpy as jnp
from jax import lax
from jax.experimental import pallas as pl
from jax.experimental.pallas import tpu as pltpu
from jax.experimental.pallas import tpu_sc as plsc

VOCAB = 100000
EMB = 300
HID = 100
OUT = 5
B = 4096
L = 200

DPAD = 128
PACK = 64
NLANE = 16
NW = 32
ROWS_PER_W = B // NW
C0, C1 = 104, 96
NSLOT = 4


def _mm_body(x_ref, w_ref, o_ref):
    t = jax.lax.dot_general(
        x_ref[...], w_ref[...],
        (((0,), (0,)), ((), ())), preferred_element_type=jnp.float32)
    lo = jax.lax.bitcast_convert_type(
        t[:, :64].astype(jnp.bfloat16), jnp.uint16).astype(jnp.uint32)
    rounded = jax.lax.bitcast_convert_type(lo[:, :28] << 16, jnp.float32)
    resid = t[:, :28] - rounded
    hi_src = jnp.concatenate([t[:, 64:HID], resid], axis=1)
    hi = jax.lax.bitcast_convert_type(
        hi_src.astype(jnp.bfloat16), jnp.uint16).astype(jnp.uint32)
    packed = jax.lax.bitcast_convert_type(lo | (hi << 16), jnp.float32)
    h = packed.shape[0] // 2
    o_ref[...] = jnp.concatenate([packed[:h], packed[h:]], axis=1)


VB = 4096
HSHIFT = VB.bit_length() - 2
NBLK = (VOCAB + VB - 1) // VB
VSLOTS = NBLK * VB


def _table_matmul(embed_t, w1pad):
    return pl.pallas_call(
        _mm_body,
        grid=(NBLK,),
        in_specs=[
            pl.BlockSpec((EMB, VB), lambda i: (0, i)),
            pl.BlockSpec((EMB, DPAD), lambda i: (0, 0)),
        ],
        out_specs=pl.BlockSpec((VB // 2, DPAD), lambda i: (i, 0)),
        out_shape=jax.ShapeDtypeStruct((VSLOTS // 2, DPAD), jnp.float32),
    )(embed_t, w1pad)


CHUNKS = ((0, 56), (56, 48), (104, 48), (152, 48))
CMAX = 56


def _sc_body(idx_hbm, tab_hbm, out_hbm, idx_v, b0, b1, b2, b3, b4, b5, b6, b7,
             out_v, s0, s1, s2, s3, s4, s5, s6, s7):
    wid = lax.axis_index("s") * 2 + lax.axis_index("c")
    base = wid * ROWS_PER_W
    pltpu.sync_copy(idx_hbm.at[pl.ds(base * L, ROWS_PER_W * L)], idx_v)
    slots = ((b0, s0), (b1, s1), (b2, s2), (b3, s3),
             (b4, s4), (b5, s5), (b6, s6), (b7, s7))

    def xform(k, carry):
        v = idx_v[pl.ds(k * NLANE, NLANE)]
        j = v & (VB - 1)
        idx_v[pl.ds(k * NLANE, NLANE)] = (
            v - j + ((j & (VB // 2 - 1)) << 1) + (j >> HSHIFT))
        return carry

    lax.fori_loop(0, ROWS_PER_W * L // NLANE, xform, 0, unroll=8)

    def fire(r, q, buf, sem):
        off, n = CHUNKS[q]
        o = pl.multiple_of(r * L + off, 8)
        pltpu.async_copy(tab_hbm.at[idx_v.at[pl.ds(o, n)]],
                         buf.at[pl.ds(0, n)], sem)

    def wait(r, q, buf, sem):
        off, n = CHUNKS[q]
        o = pl.multiple_of(r * L + off, 8)
        pltpu.make_async_copy(tab_hbm.at[idx_v.at[pl.ds(o, n)]],
                              buf.at[pl.ds(0, n)], sem).wait()

    def accum(buf, n, accs):
        def body(j, a):
            new = []
            for c in range(4):
                w = plsc.bitcast(buf[j, pl.ds(c * NLANE, NLANE)], jnp.bfloat16)
                e, o = plsc.unpack(w, format=plsc.PackFormat.INTERLEAVED,
                                   preferred_element_type=jnp.float32)
                new.append(a[c] + e)
                new.append(a[4 + c] + o)
            return tuple(new[0::2]) + tuple(new[1::2])
        return lax.fori_loop(0, n, body, accs, unroll=8)

    zero = tuple(jnp.zeros((NLANE,), jnp.float32) for _ in range(8))

    for q in range(4):
        fire(0, q, *slots[q])
        fire(1, q, *slots[4 + q])

    def outer(i, carry):
        r0 = 2 * i
        for r, s in ((r0, 0), (r0 + 1, 4)):
            accs = zero
            for q in range(4):
                buf, sem = slots[s + q]
                wait(r, q, buf, sem)
                accs = accum(buf, CHUNKS[q][1], accs)

                @pl.when(i + 1 < ROWS_PER_W // 2)
                def _():
                    fire(r + 2, q, buf, sem)

            for c in range(8):
                out_v[r, pl.ds(c * NLANE, NLANE)] = accs[c]
        return carry

    lax.fori_loop(0, ROWS_PER_W // 2, outer, 0)
    pltpu.sync_copy(out_v, out_hbm.at[pl.ds(base, ROWS_PER_W)])


def _sc_gather_sum(idx_flat, table):
    f = functools.partial(
        pl.kernel,
        out_type=jax.ShapeDtypeStruct((B, DPAD), jnp.float32),
        mesh=plsc.VectorSubcoreMesh(core_axis_name="c", subcore_axis_name="s"),
        compiler_params=pltpu.CompilerParams(use_tc_tiling_on_sc=False,
                                             needs_layout_passes=False),
        scratch_types=[
            pltpu.VMEM((ROWS_PER_W * L,), jnp.int32),
        ] + [pltpu.VMEM((CMAX, PACK), jnp.float32)] * 8 + [
            pltpu.VMEM((ROWS_PER_W, DPAD), jnp.float32),
        ] + [pltpu.SemaphoreType.DMA] * 8,
    )(_sc_body)
    return f(idx_flat, table.reshape(VSLOTS, PACK))


def _mlp_body(s_ref, b1_ref, w2_ref, b2_ref, w3_ref, b3_ref, o_ref):
    s = s_ref[...]
    s = jnp.concatenate([s[:, :28] + s[:, 100:], s[:, 28:100],
                         jnp.zeros_like(s[:, :28])], axis=1)
    h = jnp.tanh(s + b1_ref[...])
    h = jnp.tanh(jnp.dot(h, w2_ref[...], preferred_element_type=jnp.float32)
                 + b2_ref[...])
    o_ref[...] = (jnp.dot(h, w3_ref[...], preferred_element_type=jnp.float32)
                  + b3_ref[...])


def _mlp(s, b1p, w2p, b2p, w3p, b3p):
    bb = 512
    return pl.pallas_call(
        _mlp_body,
        grid=(B // bb,),
        in_specs=[
            pl.BlockSpec((bb, DPAD), lambda i: (i, 0)),
            pl.BlockSpec((1, DPAD), lambda i: (0, 0)),
            pl.BlockSpec((DPAD, DPAD), lambda i: (0, 0)),
            pl.BlockSpec((1, DPAD), lambda i: (0, 0)),
            pl.BlockSpec((DPAD, OUT), lambda i: (0, 0)),
            pl.BlockSpec((1, OUT), lambda i: (0, 0)),
        ],
        out_specs=pl.BlockSpec((bb, OUT), lambda i: (i, 0)),
        out_shape=jax.ShapeDtypeStruct((B, OUT), jnp.float32),
    )(s, b1p, w2p, b2p, w3p, b3p)


def kernel(inputs, embed, W1, b1, W2, b2, W3, b3):
    w1p = jnp.zeros((EMB, DPAD), jnp.float32).at[:, :HID].set(W1)
    table = _table_matmul(embed.T, w1p)
    s = _sc_gather_sum(inputs.reshape(-1), table)
    b1p = jnp.zeros((1, DPAD), jnp.float32).at[0, :HID].set(b1)
    w2p = jnp.zeros((DPAD, DPAD), jnp.float32).at[:HID, :HID].set(W2)
    b2p = jnp.zeros((1, DPAD), jnp.float32).at[0, :HID].set(b2)
    w3p = jnp.zeros((DPAD, OUT), jnp.float32).at[:HID, :].set(W3)
    b3p = b3.reshape(1, OUT)
    return _mlp(s, b1p, w2p, b2p, w3p, b3p)


# speedup vs baseline: 1.0937x; 1.0239x over previous
import functools

import jax
import jax.numpy as jnp
from jax import lax
from jax.experimental import pallas as pl
from jax.experimental.pallas import tpu as pltpu
from jax.experimental.pallas import tpu_sc as plsc

VOCAB = 100000
EMB = 300
HID = 100
OUT = 5
B = 4096
L = 200

DPAD = 128
PACK = 64
NLANE = 16
NW = 32
ROWS_PER_W = B // NW
C0, C1 = 104, 96
NSLOT = 4


def _mm_body(x_ref, w_ref, o_ref):
    t = jax.lax.dot_general(
        x_ref[...], w_ref[...],
        (((0,), (0,)), ((), ())), preferred_element_type=jnp.float32)
    lo = jax.lax.bitcast_convert_type(
        t[:, :64].astype(jnp.bfloat16), jnp.uint16).astype(jnp.uint32)
    rounded = jax.lax.bitcast_convert_type(lo[:, :28] << 16, jnp.float32)
    resid = t[:, :28] - rounded
    hi_src = jnp.concatenate([t[:, 64:HID], resid], axis=1)
    hi = jax.lax.bitcast_convert_type(
        hi_src.astype(jnp.bfloat16), jnp.uint16).astype(jnp.uint32)
    packed = jax.lax.bitcast_convert_type(lo | (hi << 16), jnp.float32)
    h = packed.shape[0] // 2
    o_ref[...] = jnp.concatenate([packed[:h], packed[h:]], axis=1)


VB = 8192
HSHIFT = VB.bit_length() - 2
NBLK = (VOCAB + VB - 1) // VB
VSLOTS = NBLK * VB


def _table_matmul(embed_t, w1pad):
    return pl.pallas_call(
        _mm_body,
        grid=(NBLK,),
        in_specs=[
            pl.BlockSpec((EMB, VB), lambda i: (0, i)),
            pl.BlockSpec((EMB, DPAD), lambda i: (0, 0)),
        ],
        out_specs=pl.BlockSpec((VB // 2, DPAD), lambda i: (i, 0)),
        out_shape=jax.ShapeDtypeStruct((VSLOTS // 2, DPAD), jnp.float32),
    )(embed_t, w1pad)


CHUNKS = ((0, 56), (56, 48), (104, 48), (152, 48))
CMAX = 56


def _sc_body(idx_hbm, tab_hbm, out_hbm, idx_v, b0, b1, b2, b3, b4, b5, b6, b7,
             out_v, s0, s1, s2, s3, s4, s5, s6, s7):
    wid = lax.axis_index("s") * 2 + lax.axis_index("c")
    base = wid * ROWS_PER_W
    pltpu.sync_copy(idx_hbm.at[pl.ds(base * L, ROWS_PER_W * L)], idx_v)
    slots = ((b0, s0), (b1, s1), (b2, s2), (b3, s3),
             (b4, s4), (b5, s5), (b6, s6), (b7, s7))

    def xform(k, carry):
        v = idx_v[pl.ds(k * NLANE, NLANE)]
        j = v & (VB - 1)
        idx_v[pl.ds(k * NLANE, NLANE)] = (
            v - j + ((j & (VB // 2 - 1)) << 1) + (j >> HSHIFT))
        return carry

    lax.fori_loop(0, ROWS_PER_W * L // NLANE, xform, 0, unroll=8)

    def fire(r, q, buf, sem):
        off, n = CHUNKS[q]
        o = pl.multiple_of(r * L + off, 8)
        pltpu.async_copy(tab_hbm.at[idx_v.at[pl.ds(o, n)]],
                         buf.at[pl.ds(0, n)], sem)

    def wait(r, q, buf, sem):
        off, n = CHUNKS[q]
        o = pl.multiple_of(r * L + off, 8)
        pltpu.make_async_copy(tab_hbm.at[idx_v.at[pl.ds(o, n)]],
                              buf.at[pl.ds(0, n)], sem).wait()

    def accum(buf, n, accs):
        def body(j, a):
            new = []
            for c in range(4):
                w = plsc.bitcast(buf[j, pl.ds(c * NLANE, NLANE)], jnp.bfloat16)
                e, o = plsc.unpack(w, format=plsc.PackFormat.INTERLEAVED,
                                   preferred_element_type=jnp.float32)
                new.append(a[c] + e)
                new.append(a[4 + c] + o)
            return tuple(new[0::2]) + tuple(new[1::2])
        return lax.fori_loop(0, n, body, accs, unroll=8)

    zero = tuple(jnp.zeros((NLANE,), jnp.float32) for _ in range(8))

    for q in range(4):
        fire(0, q, *slots[q])
        fire(1, q, *slots[4 + q])

    def outer(i, carry):
        r0 = 2 * i
        for r, s in ((r0, 0), (r0 + 1, 4)):
            accs = zero
            for q in range(4):
                buf, sem = slots[s + q]
                wait(r, q, buf, sem)
                accs = accum(buf, CHUNKS[q][1], accs)

                @pl.when(i + 1 < ROWS_PER_W // 2)
                def _():
                    fire(r + 2, q, buf, sem)

            for c in range(8):
                out_v[r, pl.ds(c * NLANE, NLANE)] = accs[c]
        return carry

    lax.fori_loop(0, ROWS_PER_W // 2, outer, 0)
    pltpu.sync_copy(out_v, out_hbm.at[pl.ds(base, ROWS_PER_W)])


def _sc_gather_sum(idx_flat, table):
    f = functools.partial(
        pl.kernel,
        out_type=jax.ShapeDtypeStruct((B, DPAD), jnp.float32),
        mesh=plsc.VectorSubcoreMesh(core_axis_name="c", subcore_axis_name="s"),
        compiler_params=pltpu.CompilerParams(use_tc_tiling_on_sc=False,
                                             needs_layout_passes=False),
        scratch_types=[
            pltpu.VMEM((ROWS_PER_W * L,), jnp.int32),
        ] + [pltpu.VMEM((CMAX, PACK), jnp.float32)] * 8 + [
            pltpu.VMEM((ROWS_PER_W, DPAD), jnp.float32),
        ] + [pltpu.SemaphoreType.DMA] * 8,
    )(_sc_body)
    return f(idx_flat, table.reshape(VSLOTS, PACK))


def _mlp_body(s_ref, b1_ref, w2_ref, b2_ref, w3_ref, b3_ref, o_ref):
    s = s_ref[...]
    s = jnp.concatenate([s[:, :28] + s[:, 100:], s[:, 28:100],
                         jnp.zeros_like(s[:, :28])], axis=1)
    h = jnp.tanh(s + b1_ref[...])
    h = jnp.tanh(jnp.dot(h, w2_ref[...], preferred_element_type=jnp.float32)
                 + b2_ref[...])
    o_ref[...] = (jnp.dot(h, w3_ref[...], preferred_element_type=jnp.float32)
                  + b3_ref[...])


def _mlp(s, b1p, w2p, b2p, w3p, b3p):
    bb = 512
    return pl.pallas_call(
        _mlp_body,
        grid=(B // bb,),
        in_specs=[
            pl.BlockSpec((bb, DPAD), lambda i: (i, 0)),
            pl.BlockSpec((1, DPAD), lambda i: (0, 0)),
            pl.BlockSpec((DPAD, DPAD), lambda i: (0, 0)),
            pl.BlockSpec((1, DPAD), lambda i: (0, 0)),
            pl.BlockSpec((DPAD, OUT), lambda i: (0, 0)),
            pl.BlockSpec((1, OUT), lambda i: (0, 0)),
        ],
        out_specs=pl.BlockSpec((bb, OUT), lambda i: (i, 0)),
        out_shape=jax.ShapeDtypeStruct((B, OUT), jnp.float32),
    )(s, b1p, w2p, b2p, w3p, b3p)


def kernel(inputs, embed, W1, b1, W2, b2, W3, b3):
    w1p = jnp.zeros((EMB, DPAD), jnp.float32).at[:, :HID].set(W1)
    table = _table_matmul(embed.T, w1p)
    s = _sc_gather_sum(inputs.reshape(-1), table)
    b1p = jnp.zeros((1, DPAD), jnp.float32).at[0, :HID].set(b1)
    w2p = jnp.zeros((DPAD, DPAD), jnp.float32).at[:HID, :HID].set(W2)
    b2p = jnp.zeros((1, DPAD), jnp.float32).at[0, :HID].set(b2)
    w3p = jnp.zeros((DPAD, OUT), jnp.float32).at[:HID, :].set(W3)
    b3p = b3.reshape(1, OUT)
    return _mlp(s, b1p, w2p, b2p, w3p, b3p)


# speedup vs baseline: 1.0979x; 1.0038x over previous
import functools

import jax
import jax.numpy as jnp
from jax import lax
from jax.experimental import pallas as pl
from jax.experimental.pallas import tpu as pltpu
from jax.experimental.pallas import tpu_sc as plsc

VOCAB = 100000
EMB = 300
HID = 100
OUT = 5
B = 4096
L = 200

DPAD = 128
PACK = 64
NLANE = 16
NW = 32
ROWS_PER_W = B // NW
C0, C1 = 104, 96
NSLOT = 4


def _mm_body(x_ref, w_ref, o_ref):
    t = jax.lax.dot_general(
        x_ref[...], w_ref[...],
        (((0,), (0,)), ((), ())), preferred_element_type=jnp.float32)
    lo = jax.lax.bitcast_convert_type(
        t[:, :64].astype(jnp.bfloat16), jnp.uint16).astype(jnp.uint32)
    rounded = jax.lax.bitcast_convert_type(lo[:, :28] << 16, jnp.float32)
    resid = t[:, :28] - rounded
    hi_src = jnp.concatenate([t[:, 64:HID], resid], axis=1)
    hi = jax.lax.bitcast_convert_type(
        hi_src.astype(jnp.bfloat16), jnp.uint16).astype(jnp.uint32)
    packed = jax.lax.bitcast_convert_type(lo | (hi << 16), jnp.float32)
    h = packed.shape[0] // 2
    o_ref[...] = jnp.concatenate([packed[:h], packed[h:]], axis=1)


VB = 8192
HSHIFT = VB.bit_length() - 2
NBLK = (VOCAB + VB - 1) // VB
VSLOTS = NBLK * VB


def _table_matmul(embed_t, w1pad):
    return pl.pallas_call(
        _mm_body,
        grid=(NBLK,),
        in_specs=[
            pl.BlockSpec((EMB, VB), lambda i: (0, i)),
            pl.BlockSpec((EMB, DPAD), lambda i: (0, 0)),
        ],
        out_specs=pl.BlockSpec((VB // 2, DPAD), lambda i: (i, 0)),
        out_shape=jax.ShapeDtypeStruct((VSLOTS // 2, DPAD), jnp.float32),
    )(embed_t, w1pad)


CHUNKS = ((0, 104), (104, 96))
CMAX = 104
BAGS_IN_FLIGHT = 4


def _sc_body(idx_hbm, tab_hbm, out_hbm, idx_v, b0, b1, b2, b3, b4, b5, b6, b7,
             out_v, s0, s1, s2, s3, s4, s5, s6, s7):
    wid = lax.axis_index("s") * 2 + lax.axis_index("c")
    base = wid * ROWS_PER_W
    pltpu.sync_copy(idx_hbm.at[pl.ds(base * L, ROWS_PER_W * L)], idx_v)
    slots = ((b0, s0), (b1, s1), (b2, s2), (b3, s3),
             (b4, s4), (b5, s5), (b6, s6), (b7, s7))

    def xform(k, carry):
        v = idx_v[pl.ds(k * NLANE, NLANE)]
        j = v & (VB - 1)
        idx_v[pl.ds(k * NLANE, NLANE)] = (
            v - j + ((j & (VB // 2 - 1)) << 1) + (j >> HSHIFT))
        return carry

    lax.fori_loop(0, ROWS_PER_W * L // NLANE, xform, 0, unroll=8)

    def fire(r, q, buf, sem):
        off, n = CHUNKS[q]
        o = pl.multiple_of(r * L + off, 8)
        pltpu.async_copy(tab_hbm.at[idx_v.at[pl.ds(o, n)]],
                         buf.at[pl.ds(0, n)], sem)

    def wait(r, q, buf, sem):
        off, n = CHUNKS[q]
        o = pl.multiple_of(r * L + off, 8)
        pltpu.make_async_copy(tab_hbm.at[idx_v.at[pl.ds(o, n)]],
                              buf.at[pl.ds(0, n)], sem).wait()

    def accum(buf, n, accs):
        def body(j, a):
            new = []
            for c in range(4):
                w = plsc.bitcast(buf[j, pl.ds(c * NLANE, NLANE)], jnp.bfloat16)
                e, o = plsc.unpack(w, format=plsc.PackFormat.INTERLEAVED,
                                   preferred_element_type=jnp.float32)
                new.append(a[c] + e)
                new.append(a[4 + c] + o)
            return tuple(new[0::2]) + tuple(new[1::2])
        return lax.fori_loop(0, n, body, accs, unroll=8)

    zero = tuple(jnp.zeros((NLANE,), jnp.float32) for _ in range(8))

    for m in range(BAGS_IN_FLIGHT):
        for q in range(2):
            fire(m, q, *slots[2 * m + q])

    n_iter = ROWS_PER_W // BAGS_IN_FLIGHT

    def outer(i, carry):
        r0 = BAGS_IN_FLIGHT * i
        for m in range(BAGS_IN_FLIGHT):
            r = r0 + m
            accs = zero
            for q in range(2):
                buf, sem = slots[2 * m + q]
                wait(r, q, buf, sem)
                accs = accum(buf, CHUNKS[q][1], accs)

                @pl.when(i + 1 < n_iter)
                def _():
                    fire(r + BAGS_IN_FLIGHT, q, buf, sem)

            for c in range(8):
                out_v[r, pl.ds(c * NLANE, NLANE)] = accs[c]
        return carry

    lax.fori_loop(0, n_iter, outer, 0)
    pltpu.sync_copy(out_v, out_hbm.at[pl.ds(base, ROWS_PER_W)])


def _sc_gather_sum(idx_flat, table):
    f = functools.partial(
        pl.kernel,
        out_type=jax.ShapeDtypeStruct((B, DPAD), jnp.float32),
        mesh=plsc.VectorSubcoreMesh(core_axis_name="c", subcore_axis_name="s"),
        compiler_params=pltpu.CompilerParams(use_tc_tiling_on_sc=False,
                                             needs_layout_passes=False),
        scratch_types=[
            pltpu.VMEM((ROWS_PER_W * L,), jnp.int32),
        ] + [pltpu.VMEM((CMAX, PACK), jnp.float32)] * 8 + [
            pltpu.VMEM((ROWS_PER_W, DPAD), jnp.float32),
        ] + [pltpu.SemaphoreType.DMA] * 8,
    )(_sc_body)
    return f(idx_flat, table.reshape(VSLOTS, PACK))


def _mlp_body(s_ref, b1_ref, w2_ref, b2_ref, w3_ref, b3_ref, o_ref):
    s = s_ref[...]
    s = jnp.concatenate([s[:, :28] + s[:, 100:], s[:, 28:100],
                         jnp.zeros_like(s[:, :28])], axis=1)
    h = jnp.tanh(s + b1_ref[...])
    h = jnp.tanh(jnp.dot(h, w2_ref[...], preferred_element_type=jnp.float32)
                 + b2_ref[...])
    o_ref[...] = (jnp.dot(h, w3_ref[...], preferred_element_type=jnp.float32)
                  + b3_ref[...])


def _mlp(s, b1p, w2p, b2p, w3p, b3p):
    bb = 512
    return pl.pallas_call(
        _mlp_body,
        grid=(B // bb,),
        in_specs=[
            pl.BlockSpec((bb, DPAD), lambda i: (i, 0)),
            pl.BlockSpec((1, DPAD), lambda i: (0, 0)),
            pl.BlockSpec((DPAD, DPAD), lambda i: (0, 0)),
            pl.BlockSpec((1, DPAD), lambda i: (0, 0)),
            pl.BlockSpec((DPAD, OUT), lambda i: (0, 0)),
            pl.BlockSpec((1, OUT), lambda i: (0, 0)),
        ],
        out_specs=pl.BlockSpec((bb, OUT), lambda i: (i, 0)),
        out_shape=jax.ShapeDtypeStruct((B, OUT), jnp.float32),
    )(s, b1p, w2p, b2p, w3p, b3p)


def kernel(inputs, embed, W1, b1, W2, b2, W3, b3):
    w1p = jnp.zeros((EMB, DPAD), jnp.float32).at[:, :HID].set(W1)
    table = _table_matmul(embed.T, w1p)
    s = _sc_gather_sum(inputs.reshape(-1), table)
    b1p = jnp.zeros((1, DPAD), jnp.float32).at[0, :HID].set(b1)
    w2p = jnp.zeros((DPAD, DPAD), jnp.float32).at[:HID, :HID].set(W2)
    b2p = jnp.zeros((1, DPAD), jnp.float32).at[0, :HID].set(b2)
    w3p = jnp.zeros((DPAD, OUT), jnp.float32).at[:HID, :].set(W3)
    b3p = b3.reshape(1, OUT)
    return _mlp(s, b1p, w2p, b2p, w3p, b3p)
